# Initial kernel scaffold; baseline (speedup 1.0000x reference)
import jax
import jax.numpy as jnp
from jax.experimental import pallas as pl


def kernel(token_ids, weight):
    raise NotImplementedError("write your pallas kernel here")


# speedup vs baseline: 1.8756x; 1.8756x over previous
import functools

import jax
import jax.numpy as jnp
from jax import lax
from jax.experimental import pallas as pl
from jax.experimental.pallas import tpu as pltpu
from jax.experimental.pallas import tpu_sc as plsc

C = 128
K = 4
NPAR = 2
NC = 2
NS = 16
NW = NC * NS


@functools.lru_cache(maxsize=None)
def _build(B, D):
    assert B % (NW * C * K * NPAR) == 0
    b_per_w = B // NW
    nchunks = b_per_w // C
    G = K * C
    NG = nchunks // K
    mesh = plsc.VectorSubcoreMesh(
        core_axis_name="c", subcore_axis_name="s", num_cores=NC, num_subcores=NS
    )

    @functools.partial(
        pl.kernel,
        mesh=mesh,
        out_type=jax.ShapeDtypeStruct((B, D), jnp.float32),
        scratch_types=[
            pltpu.VMEM((nchunks, C), jnp.int32),
            pltpu.VMEM((G, D), jnp.float32),
            pltpu.VMEM((G, D), jnp.float32),
            pltpu.SemaphoreType.DMA,
            pltpu.SemaphoreType.DMA,
        ],
        compiler_params=pltpu.CompilerParams(use_tc_tiling_on_sc=False),
    )
    def emb(idx_hbm, tab_hbm, out_hbm, idx_v, buf0, buf1, sem0, sem1):
        wid = lax.axis_index("s") * NC + lax.axis_index("c")
        base = wid * b_per_w
        pltpu.sync_copy(idx_hbm.at[wid], idx_v)
        bufs = (buf0, buf1)
        sems = (sem0, sem1)

        def fire(s, p):
            for k in range(K):
                pltpu.async_copy(
                    tab_hbm.at[idx_v.at[s * K + k]],
                    bufs[p].at[pl.ds(k * C, C)],
                    sems[p],
                )

        def drain(p):
            pltpu.make_async_copy(
                out_hbm.at[pl.ds(base, G)], bufs[p], sems[p]
            ).wait()

        def flush(s, p):
            pltpu.sync_copy(bufs[p], out_hbm.at[pl.ds(base + s * G, G)])

        for p in range(NPAR):
            fire(p, p)

        def step(t, carry):
            for p in range(NPAR):
                s = t * NPAR + p
                drain(p)
                flush(s, p)
                fire(s + NPAR, p)
            return carry

        lax.fori_loop(0, NG // NPAR - 1, step, 0)

        for p in range(NPAR):
            s = NG - NPAR + p
            drain(p)
            flush(s, p)

    return emb


def kernel(token_ids, weight):
    lead_shape = token_ids.shape
    B = 1
    for d in lead_shape:
        B *= d
    D = weight.shape[1]
    emb = _build(B, D)
    idx = token_ids.astype(jnp.int32).reshape(NW, B // (NW * C), C)
    out = emb(idx, weight)
    return out.reshape(lead_shape + (D,))
